# baseline (speedup 1.0000x reference)
import functools

import jax
import jax.numpy as jnp
from jax.experimental import pallas as pl
from jax.experimental.pallas import tpu as pltpu

_OMEGA0 = 30.0
_A = 16.0
_LANES = 128

_INV_2PI = 0.15915494309189535
_2PI_HI = 6.28125
_2PI_LO = 1.9353071795864769e-03
_S1 = -0.16620608652413862
_S2 = 0.008067752228579379
_S3 = -0.00015163412998563905


def _cheap_sin(arg):
    kf = jnp.round(arg * _INV_2PI)
    r = arg - kf * _2PI_HI
    r = r - kf * _2PI_LO
    r2 = r * r
    p = (_S3 * r2 + _S2) * r2 + _S1
    return r + r * (r2 * p)


def _loe_kernel(coords_ref, feats_ref, wall_ref, mb_ref, e_ref, comp_ref,
                o_ref, *, Cout, chunk):
    e = e_ref[...]
    for j in range(o_ref.shape[0] // chunk):
        sl = pl.ds(j * chunk, chunk)
        x = feats_ref[sl, :]
        y = jnp.dot(x, wall_ref[...], preferred_element_type=jnp.float32)

        pf = (jnp.floor(coords_ref[sl, :] * _A).astype(jnp.int32) & 1
              ).astype(jnp.float32)
        tile_b = jnp.dot(pf, mb_ref[...], preferred_element_type=jnp.float32)

        ym = jnp.where(tile_b == e, y, 0.0)

        s = _cheap_sin(ym)
        o_ref[sl, :] = jnp.dot(s, comp_ref[...],
                               preferred_element_type=jnp.float32)


def kernel(in_feats, in_coords, weights):
    B, Cin = in_feats.shape
    N, _, Cout = weights.shape

    wall = (jnp.float32(_OMEGA0) * weights.astype(jnp.float32)
            ).transpose(1, 0, 2).reshape(Cin, N * Cout)
    comp = jnp.tile(jnp.eye(Cout, dtype=jnp.float32), (N, 1))
    mb = jnp.stack([jnp.full((_LANES,), 2.0, jnp.float32),
                    jnp.full((_LANES,), 1.0, jnp.float32)])
    e_lane = jnp.repeat(jnp.arange(N, dtype=jnp.float32), Cout)[None, :]

    rb = min(16384, B)
    n_steps = B // rb

    out = pl.pallas_call(
        functools.partial(_loe_kernel, Cout=Cout, chunk=min(512, rb)),
        out_shape=jax.ShapeDtypeStruct((B, Cout), jnp.float32),
        grid=(n_steps,),
        in_specs=[
            pl.BlockSpec((rb, 2), lambda i: (i, 0)),
            pl.BlockSpec((rb, Cin), lambda i: (i, 0)),
            pl.BlockSpec((Cin, N * Cout), lambda i: (0, 0)),
            pl.BlockSpec((2, _LANES), lambda i: (0, 0)),
            pl.BlockSpec((1, _LANES), lambda i: (0, 0)),
            pl.BlockSpec((N * Cout, Cout), lambda i: (0, 0)),
        ],
        out_specs=pl.BlockSpec((rb, Cout), lambda i: (i, 0)),
        compiler_params=pltpu.CompilerParams(
            dimension_semantics=("parallel",),
            vmem_limit_bytes=64 * 1024 * 1024),
    )(in_coords, in_feats, wall, mb, e_lane, comp)

    return out, in_coords


# speedup vs baseline: 1.0435x; 1.0435x over previous
import functools

import jax
import jax.numpy as jnp
from jax.experimental import pallas as pl
from jax.experimental.pallas import tpu as pltpu

_OMEGA0 = 30.0
_A = 16.0
_LANES = 128

_INV_2PI = 0.15915494309189535
_2PI_HI = 6.28125
_2PI_LO = 1.9353071795864769e-03
_S1 = -0.16620608652413862
_S2 = 0.008067752228579379
_S3 = -0.00015163412998563905


def _cheap_sin(arg):
    kf = jnp.round(arg * _INV_2PI)
    r = arg - kf * _2PI_HI
    r = r - kf * _2PI_LO
    r2 = r * r
    p = (_S3 * r2 + _S2) * r2 + _S1
    return r + r * (r2 * p)


def _loe_kernel(coords_ref, feats_ref, wall_ref, mb_ref, e_ref, comp_ref,
                o_ref, *, Cout, chunk):
    e = e_ref[...]
    for j in range(o_ref.shape[0] // chunk):
        sl = pl.ds(j * chunk, chunk)
        x = feats_ref[sl, :]
        y = jnp.dot(x, wall_ref[...], preferred_element_type=jnp.float32)

        pf = (jnp.floor(coords_ref[sl, :] * _A).astype(jnp.int32) & 1
              ).astype(jnp.float32)
        tile_b = jnp.dot(pf, mb_ref[...], preferred_element_type=jnp.float32)

        ym = jnp.where(tile_b == e, y, 0.0)

        s = _cheap_sin(ym)
        o_ref[sl, :] = jnp.dot(s, comp_ref[...],
                               preferred_element_type=jnp.float32)


def kernel(in_feats, in_coords, weights):
    B, Cin = in_feats.shape
    N, _, Cout = weights.shape

    wall = (jnp.float32(_OMEGA0) * weights.astype(jnp.float32)
            ).transpose(1, 0, 2).reshape(Cin, N * Cout)
    comp = jnp.tile(jnp.eye(Cout, dtype=jnp.float32), (N, 1))
    mb = jnp.stack([jnp.full((_LANES,), 2.0, jnp.float32),
                    jnp.full((_LANES,), 1.0, jnp.float32)])
    e_lane = jnp.repeat(jnp.arange(N, dtype=jnp.float32), Cout)[None, :]

    rb = min(16384, B)
    n_steps = B // rb

    out = pl.pallas_call(
        functools.partial(_loe_kernel, Cout=Cout, chunk=min(1024, rb)),
        out_shape=jax.ShapeDtypeStruct((B, Cout), jnp.float32),
        grid=(n_steps,),
        in_specs=[
            pl.BlockSpec((rb, 2), lambda i: (i, 0)),
            pl.BlockSpec((rb, Cin), lambda i: (i, 0)),
            pl.BlockSpec((Cin, N * Cout), lambda i: (0, 0)),
            pl.BlockSpec((2, _LANES), lambda i: (0, 0)),
            pl.BlockSpec((1, _LANES), lambda i: (0, 0)),
            pl.BlockSpec((N * Cout, Cout), lambda i: (0, 0)),
        ],
        out_specs=pl.BlockSpec((rb, Cout), lambda i: (i, 0)),
        compiler_params=pltpu.CompilerParams(
            dimension_semantics=("parallel",),
            vmem_limit_bytes=64 * 1024 * 1024),
    )(in_coords, in_feats, wall, mb, e_lane, comp)

    return out, in_coords
